# Initial kernel scaffold; baseline (speedup 1.0000x reference)
import jax
import jax.numpy as jnp
from jax.experimental import pallas as pl


def kernel(x, edge_index, edge_attr, batch, atom_emb, W_edge, b_edge, W1, b1, W2, b2, W_ih_gru, W_hh_gru, b_ih_gru, b_hh_gru, W_ih_lstm, W_hh_lstm, b_ih_lstm, b_hh_lstm, W_fc, b_fc):
    raise NotImplementedError("write your pallas kernel here")


# speedup vs baseline: 2.9225x; 2.9225x over previous
import functools

import jax
import jax.numpy as jnp
from jax import lax
from jax.experimental import pallas as pl
from jax.experimental.pallas import tpu as pltpu
from jax.experimental.pallas import tpu_sc as plsc

N = 10000
E = 320000
H = 128
B = 64
NUM_FEAT = 9
VOCAB = 64
EPS = 1e-10

NC = 2
NS = 16
NW = NC * NS
EPW = E // NW
W = 80
NWIN = EPW // W
NPAD = 10240
RPT = NPAD // NS
ZR = 128

FP32 = jnp.float32


def _encode_body(x_ref, emb_ref, o_ref):
    x = x_ref[...]
    iota = lax.broadcasted_iota(jnp.int32, (1, VOCAB), 1)
    acc = jnp.zeros((x.shape[0], H), FP32)
    for f in range(NUM_FEAT):
        oh = (x[:, f:f + 1] == iota).astype(FP32)
        acc = acc + jnp.dot(oh, emb_ref[f], preferred_element_type=FP32)
    o_ref[...] = acc


def _encode(x, atom_emb):
    return pl.pallas_call(
        _encode_body,
        out_shape=jax.ShapeDtypeStruct((N, H), FP32),
    )(x, atom_emb)


EB = 2000


def _edge_body(ea_ref, w_ref, b_ref, o_ref):
    o_ref[...] = (
        jnp.dot(ea_ref[...], w_ref[...], preferred_element_type=FP32)
        + b_ref[...]
    )


def _edge_mlp(edge_attr, W_edge, b_edge):
    return pl.pallas_call(
        _edge_body,
        grid=(E // EB,),
        in_specs=[
            pl.BlockSpec((EB, 16), lambda i: (i, 0)),
            pl.BlockSpec((16, H), lambda i: (0, 0)),
            pl.BlockSpec((1, H), lambda i: (0, 0)),
        ],
        out_specs=pl.BlockSpec((EB, H), lambda i: (i, 0)),
        out_shape=jax.ShapeDtypeStruct((E, H), FP32),
    )(edge_attr, W_edge, b_edge)


RB = 1000


def _dense_body(h_ref, p0_ref, p1_ref, w1_ref, b1_ref, w2_ref, b2_ref,
                wih_ref, bih_ref, whh_ref, bhh_ref, o_ref):
    h = h_ref[...]
    z = h + p0_ref[...] + p1_ref[...]
    t = jnp.maximum(
        jnp.dot(z, w1_ref[...], preferred_element_type=FP32) + b1_ref[...], 0.0)
    z2 = jnp.dot(t, w2_ref[...], preferred_element_type=FP32) + b2_ref[...]
    gx = jnp.dot(z2, wih_ref[...], preferred_element_type=FP32) + bih_ref[...]
    gh = jnp.dot(h, whh_ref[...], preferred_element_type=FP32) + bhh_ref[...]
    r = jax.nn.sigmoid(gx[:, :H] + gh[:, :H])
    zg = jax.nn.sigmoid(gx[:, H:2 * H] + gh[:, H:2 * H])
    n = jnp.tanh(gx[:, 2 * H:] + r * gh[:, 2 * H:])
    o_ref[...] = (1.0 - zg) * n + zg * h


def _dense_layer(h, p0, p1, W1, b1, W2, b2, WihT, b_ih, WhhT, b_hh):
    wspec = lambda shape: pl.BlockSpec(shape, lambda i: (0, 0))
    return pl.pallas_call(
        _dense_body,
        grid=(N // RB,),
        in_specs=[
            pl.BlockSpec((RB, H), lambda i: (i, 0)),
            pl.BlockSpec((RB, H), lambda i: (i, 0)),
            pl.BlockSpec((RB, H), lambda i: (i, 0)),
            wspec((H, H)), wspec((1, H)),
            wspec((H, H)), wspec((1, H)),
            wspec((H, 3 * H)), wspec((1, 3 * H)),
            wspec((H, 3 * H)), wspec((1, 3 * H)),
        ],
        out_specs=pl.BlockSpec((RB, H), lambda i: (i, 0)),
        out_shape=jax.ShapeDtypeStruct((N, H), FP32),
    )(h, p0, p1, W1, b1, W2, b2, WihT, b_ih, WhhT, b_hh)


def _s2s_body(h_ref, bt_ref, wih_ref, bih_ref, whh_ref, bhh_ref,
              wfc_ref, bfc_ref, o_ref):
    h = h_ref[...]
    bt = bt_ref[...]
    mask = (bt == lax.broadcasted_iota(jnp.int32, (1, B), 1)).astype(FP32)
    hl = jnp.zeros((B, H), FP32)
    cl = jnp.zeros((B, H), FP32)
    q_star = jnp.zeros((B, 2 * H), FP32)
    for _ in range(3):
        gates = (
            jnp.dot(q_star, wih_ref[...], preferred_element_type=FP32)
            + bih_ref[...]
            + jnp.dot(hl, whh_ref[...], preferred_element_type=FP32)
            + bhh_ref[...]
        )
        ig = jax.nn.sigmoid(gates[:, :H])
        fg = jax.nn.sigmoid(gates[:, H:2 * H])
        gg = jnp.tanh(gates[:, 2 * H:3 * H])
        og = jax.nn.sigmoid(gates[:, 3 * H:])
        cl = fg * cl + ig * gg
        hl = og * jnp.tanh(cl)
        p_all = lax.dot_general(h, hl, (((1,), (1,)), ((), ())),
                                preferred_element_type=FP32)
        prod = jnp.sum(mask * p_all, axis=1, keepdims=True)
        m = jnp.max(jnp.where(mask > 0, p_all, -1e30), axis=0, keepdims=True)
        e = jnp.exp(prod - jnp.sum(mask * m, axis=1, keepdims=True))
        norm = jnp.sum(mask * e, axis=0, keepdims=True)
        att = e / (jnp.sum(mask * norm, axis=1, keepdims=True) + EPS)
        att_out = lax.dot_general(mask * att, h, (((0,), (0,)), ((), ())),
                                  preferred_element_type=FP32)
        q_star = jnp.concatenate([hl, att_out], axis=1)
    o_ref[...] = (
        jnp.dot(q_star, wfc_ref[...], preferred_element_type=FP32)
        + bfc_ref[...]
    )


def _set2set(h, batch2d, WihT, b_ih, WhhT, b_hh, W_fc, b_fc):
    return pl.pallas_call(
        _s2s_body,
        out_shape=jax.ShapeDtypeStruct((B, 1), FP32),
    )(h, batch2d, WihT, b_ih, WhhT, b_hh, W_fc, b_fc)


_SC_MESH = plsc.VectorSubcoreMesh(core_axis_name="c", subcore_axis_name="s")


def _sc_body(h_hbm, eh_hbm, src_hbm, dst_hbm, out_hbm,
             srcb, dstb, hrows, erows, msg, zbuf, accum, sem):
    cid = lax.axis_index("c")
    sid = lax.axis_index("s")
    wid = cid * NS + sid

    def zrow(r, carry):
        for c in range(8):
            zbuf[r, pl.ds(c * 16, 16)] = jnp.zeros((16,), FP32)
        return carry

    lax.fori_loop(0, ZR, zrow, 0)
    for k in range(RPT // ZR):
        pltpu.sync_copy(zbuf, accum.at[pl.ds(sid * RPT + k * ZR, ZR)])
    plsc.subcore_barrier()

    ebase = wid * EPW

    def win(i, carry):
        base = ebase + i * W
        pltpu.sync_copy(src_hbm.at[pl.ds(base, W)], srcb)
        pltpu.sync_copy(dst_hbm.at[pl.ds(base, W)], dstb)
        pltpu.async_copy(h_hbm.at[srcb], hrows, sem).wait()
        pltpu.sync_copy(eh_hbm.at[pl.ds(base, W)], erows)

        def erow(e2, carry2):
            for c in range(8):
                hv = hrows[e2, pl.ds(c * 16, 16)]
                ev = erows[e2, pl.ds(c * 16, 16)]
                msg[e2, pl.ds(c * 16, 16)] = jnp.maximum(hv + ev, 0.0)
            return carry2

        lax.fori_loop(0, W, erow, 0)
        pltpu.sync_copy(msg, accum.at[dstb], add=True)
        return carry

    lax.fori_loop(0, NWIN, win, 0)
    plsc.subcore_barrier()
    for k in range(RPT // ZR):
        rbase = sid * RPT + k * ZR
        pltpu.sync_copy(accum.at[pl.ds(rbase, ZR)],
                        out_hbm.at[cid, pl.ds(rbase, ZR)])


@functools.partial(
    pl.kernel,
    out_type=jax.ShapeDtypeStruct((NC, NPAD, H), FP32),
    mesh=_SC_MESH,
    scratch_types=[
        pltpu.VMEM((W,), jnp.int32),
        pltpu.VMEM((W,), jnp.int32),
        pltpu.VMEM((W, H), FP32),
        pltpu.VMEM((W, H), FP32),
        pltpu.VMEM((W, H), FP32),
        pltpu.VMEM((ZR, H), FP32),
        pltpu.VMEM_SHARED((NPAD, H), FP32),
        pltpu.SemaphoreType.DMA,
    ],
)
def _sc_layer(h_hbm, eh_hbm, src_hbm, dst_hbm, out_hbm,
              srcb, dstb, hrows, erows, msg, zbuf, accum, sem):
    _sc_body(h_hbm, eh_hbm, src_hbm, dst_hbm, out_hbm,
             srcb, dstb, hrows, erows, msg, zbuf, accum, sem)


def kernel(x, edge_index, edge_attr, batch, atom_emb, W_edge, b_edge, W1, b1,
           W2, b2, W_ih_gru, W_hh_gru, b_ih_gru, b_hh_gru,
           W_ih_lstm, W_hh_lstm, b_ih_lstm, b_hh_lstm, W_fc, b_fc):
    src = edge_index[0]
    dst = edge_index[1]
    batch2d = batch.reshape(N, 1)
    WihT_g = W_ih_gru.T
    WhhT_g = W_hh_gru.T
    WihT_l = W_ih_lstm.T
    WhhT_l = W_hh_lstm.T
    b_edge2 = b_edge.reshape(1, H)
    b1_2 = b1.reshape(1, H)
    b2_2 = b2.reshape(1, H)
    bih_g = b_ih_gru.reshape(1, 3 * H)
    bhh_g = b_hh_gru.reshape(1, 3 * H)
    bih_l = b_ih_lstm.reshape(1, 4 * H)
    bhh_l = b_hh_lstm.reshape(1, 4 * H)
    bfc2 = b_fc.reshape(1, 1)

    h = _encode(x, atom_emb)
    edge_h = _edge_mlp(edge_attr, W_edge, b_edge2)
    for _ in range(3):
        partials = _sc_layer(h, edge_h, src, dst)
        h = _dense_layer(h, partials[0], partials[1], W1, b1_2, W2, b2_2,
                         WihT_g, bih_g, WhhT_g, bhh_g)
    return _set2set(h, batch2d, WihT_l, bih_l, WhhT_l, bhh_l, W_fc, bfc2)


# speedup vs baseline: 3.8348x; 1.3122x over previous
import functools

import jax
import jax.numpy as jnp
from jax import lax
from jax.experimental import pallas as pl
from jax.experimental.pallas import tpu as pltpu
from jax.experimental.pallas import tpu_sc as plsc

N = 10000
E = 320000
H = 128
B = 64
NUM_FEAT = 9
VOCAB = 64
EPS = 1e-10

NC = 2
NS = 16
NW = NC * NS
EPW = E // NW
W = 80
NWIN = EPW // W
NPAD = 10240
RPT = NPAD // NS
ZR = 128

FP32 = jnp.float32


def _encode_body(x_ref, emb_ref, o_ref):
    x = x_ref[...]
    iota = lax.broadcasted_iota(jnp.int32, (1, VOCAB), 1)
    acc = jnp.zeros((x.shape[0], H), FP32)
    for f in range(NUM_FEAT):
        oh = (x[:, f:f + 1] == iota).astype(FP32)
        acc = acc + jnp.dot(oh, emb_ref[f], preferred_element_type=FP32)
    o_ref[...] = acc


def _encode(x, atom_emb):
    return pl.pallas_call(
        _encode_body,
        out_shape=jax.ShapeDtypeStruct((N, H), FP32),
    )(x, atom_emb)


EB = 2000


def _edge_body(ea_ref, wlo_ref, blo_ref, whi_ref, bhi_ref, o_ref):
    ea = ea_ref[...]

    def pack(sub):
        elo = (jnp.dot(sub, wlo_ref[...], preferred_element_type=FP32)
               + blo_ref[...])
        ehi = (jnp.dot(sub, whi_ref[...], preferred_element_type=FP32)
               + bhi_ref[...])
        lb = lax.bitcast_convert_type(elo, jnp.int32) + 0x8000
        hb = lax.bitcast_convert_type(ehi, jnp.int32) + 0x8000
        return (hb & jnp.int32(-65536)) | lax.shift_right_logical(lb, 16)

    o_ref[...] = jnp.concatenate(
        [pack(ea[:, :16]), pack(ea[:, 16:])], axis=1)


def _edge_mlp(edge_attr2, W_lo, b_lo, W_hi, b_hi):
    return pl.pallas_call(
        _edge_body,
        grid=(E // 2 // EB,),
        in_specs=[
            pl.BlockSpec((EB, 32), lambda i: (i, 0)),
            pl.BlockSpec((16, H // 2), lambda i: (0, 0)),
            pl.BlockSpec((1, H // 2), lambda i: (0, 0)),
            pl.BlockSpec((16, H // 2), lambda i: (0, 0)),
            pl.BlockSpec((1, H // 2), lambda i: (0, 0)),
        ],
        out_specs=pl.BlockSpec((EB, H), lambda i: (i, 0)),
        out_shape=jax.ShapeDtypeStruct((E // 2, H), jnp.int32),
    )(edge_attr2, W_lo, b_lo, W_hi, b_hi)


RB = 1000


def _dense_body(h_ref, p0_ref, p1_ref, w1_ref, b1_ref, w2_ref, b2_ref,
                wih_ref, bih_ref, whh_ref, bhh_ref, o_ref):
    h = h_ref[...]
    z = h + p0_ref[...] + p1_ref[...]
    t = jnp.maximum(
        jnp.dot(z, w1_ref[...], preferred_element_type=FP32) + b1_ref[...], 0.0)
    z2 = jnp.dot(t, w2_ref[...], preferred_element_type=FP32) + b2_ref[...]
    gx = jnp.dot(z2, wih_ref[...], preferred_element_type=FP32) + bih_ref[...]
    gh = jnp.dot(h, whh_ref[...], preferred_element_type=FP32) + bhh_ref[...]
    r = jax.nn.sigmoid(gx[:, :H] + gh[:, :H])
    zg = jax.nn.sigmoid(gx[:, H:2 * H] + gh[:, H:2 * H])
    n = jnp.tanh(gx[:, 2 * H:] + r * gh[:, 2 * H:])
    o_ref[...] = (1.0 - zg) * n + zg * h


def _dense_layer(h, p0, p1, W1, b1, W2, b2, WihT, b_ih, WhhT, b_hh):
    wspec = lambda shape: pl.BlockSpec(shape, lambda i: (0, 0))
    return pl.pallas_call(
        _dense_body,
        grid=(N // RB,),
        in_specs=[
            pl.BlockSpec((RB, H), lambda i: (i, 0)),
            pl.BlockSpec((RB, H), lambda i: (i, 0)),
            pl.BlockSpec((RB, H), lambda i: (i, 0)),
            wspec((H, H)), wspec((1, H)),
            wspec((H, H)), wspec((1, H)),
            wspec((H, 3 * H)), wspec((1, 3 * H)),
            wspec((H, 3 * H)), wspec((1, 3 * H)),
        ],
        out_specs=pl.BlockSpec((RB, H), lambda i: (i, 0)),
        out_shape=jax.ShapeDtypeStruct((N, H), FP32),
    )(h, p0, p1, W1, b1, W2, b2, WihT, b_ih, WhhT, b_hh)


def _s2s_body(h_ref, bt_ref, wih_ref, bih_ref, whh_ref, bhh_ref,
              wfc_ref, bfc_ref, o_ref):
    h = h_ref[...]
    bt = bt_ref[...]
    mask = (bt == lax.broadcasted_iota(jnp.int32, (1, B), 1)).astype(FP32)
    hl = jnp.zeros((B, H), FP32)
    cl = jnp.zeros((B, H), FP32)
    q_star = jnp.zeros((B, 2 * H), FP32)
    for _ in range(3):
        gates = (
            jnp.dot(q_star, wih_ref[...], preferred_element_type=FP32)
            + bih_ref[...]
            + jnp.dot(hl, whh_ref[...], preferred_element_type=FP32)
            + bhh_ref[...]
        )
        ig = jax.nn.sigmoid(gates[:, :H])
        fg = jax.nn.sigmoid(gates[:, H:2 * H])
        gg = jnp.tanh(gates[:, 2 * H:3 * H])
        og = jax.nn.sigmoid(gates[:, 3 * H:])
        cl = fg * cl + ig * gg
        hl = og * jnp.tanh(cl)
        p_all = lax.dot_general(h, hl, (((1,), (1,)), ((), ())),
                                preferred_element_type=FP32)
        prod = jnp.sum(mask * p_all, axis=1, keepdims=True)
        m = jnp.max(jnp.where(mask > 0, p_all, -1e30), axis=0, keepdims=True)
        e = jnp.exp(prod - jnp.sum(mask * m, axis=1, keepdims=True))
        norm = jnp.sum(mask * e, axis=0, keepdims=True)
        att = e / (jnp.sum(mask * norm, axis=1, keepdims=True) + EPS)
        att_out = lax.dot_general(mask * att, h, (((0,), (0,)), ((), ())),
                                  preferred_element_type=FP32)
        q_star = jnp.concatenate([hl, att_out], axis=1)
    o_ref[...] = (
        jnp.dot(q_star, wfc_ref[...], preferred_element_type=FP32)
        + bfc_ref[...]
    )


def _set2set(h, batch2d, WihT, b_ih, WhhT, b_hh, W_fc, b_fc):
    return pl.pallas_call(
        _s2s_body,
        out_shape=jax.ShapeDtypeStruct((B, 1), FP32),
    )(h, batch2d, WihT, b_ih, WhhT, b_hh, W_fc, b_fc)


_SC_MESH = plsc.VectorSubcoreMesh(core_axis_name="c", subcore_axis_name="s")


def _sc_body(h_hbm, eh_hbm, src_hbm, dst_hbm, out_hbm,
             srcb0, dstb0, hrows0, erows0,
             srcb1, dstb1, hrows1, erows1,
             msg, accum,
             sem_i0, sem_i1, sem_g0, sem_g1, sem_e0, sem_e1):
    cid = lax.axis_index("c")
    sid = lax.axis_index("s")
    wid = cid * NS + sid

    srcb = (srcb0, srcb1)
    dstb = (dstb0, dstb1)
    hrows = (hrows0, hrows1)
    erows = (erows0, erows1)
    sem_i = (sem_i0, sem_i1)
    sem_g = (sem_g0, sem_g1)
    sem_e = (sem_e0, sem_e1)

    def zrow(r, carry):
        for c in range(8):
            msg[r, pl.ds(c * 16, 16)] = jnp.zeros((16,), FP32)
        return carry

    lax.fori_loop(0, W, zrow, 0)
    for k in range(RPT // W):
        pltpu.sync_copy(msg, accum.at[pl.ds(sid * RPT + k * W, W)])
    plsc.subcore_barrier()

    ebase = wid * EPW

    ebase2 = wid * (EPW // 2)

    def issue_loads(i, b):
        base = ebase + i * W
        base2 = ebase2 + i * (W // 2)
        pltpu.async_copy(src_hbm.at[pl.ds(base, W)], srcb[b], sem_i[b])
        pltpu.async_copy(dst_hbm.at[pl.ds(base, W)], dstb[b], sem_i[b])
        pltpu.async_copy(eh_hbm.at[pl.ds(base2, W // 2)], erows[b],
                         sem_e[b])

    def issue_gather(b):
        pltpu.make_async_copy(src_hbm.at[pl.ds(0, W)], srcb[b], sem_i[b]).wait()
        pltpu.make_async_copy(dst_hbm.at[pl.ds(0, W)], dstb[b], sem_i[b]).wait()
        pltpu.async_copy(h_hbm.at[srcb[b]], hrows[b], sem_g[b])

    def compute_scatter(b):
        pltpu.make_async_copy(h_hbm.at[srcb[b]], hrows[b], sem_g[b]).wait()
        pltpu.make_async_copy(eh_hbm.at[pl.ds(0, W // 2)], erows[b],
                              sem_e[b]).wait()
        hr = hrows[b]
        er = erows[b]

        def erow(r2, carry2):
            for half in range(2):
                e2 = 2 * r2 + half
                for g in range(4):
                    u = er[r2, pl.ds(half * 64 + g * 16, 16)]
                    flo = plsc.bitcast(u << 16, FP32)
                    fhi = plsc.bitcast(u & jnp.int32(-65536), FP32)
                    h0 = hr[e2, pl.ds(2 * g * 16, 16)]
                    h1 = hr[e2, pl.ds((2 * g + 1) * 16, 16)]
                    msg[e2, pl.ds(2 * g * 16, 16)] = jnp.maximum(h0 + flo, 0.0)
                    msg[e2, pl.ds((2 * g + 1) * 16, 16)] = (
                        jnp.maximum(h1 + fhi, 0.0))
            return carry2

        lax.fori_loop(0, W // 2, erow, 0)
        pltpu.sync_copy(msg, accum.at[dstb[b]], add=True)

    issue_loads(0, 0)
    issue_loads(1, 1)
    issue_gather(0)

    def pair(j, carry):
        i = 2 * j
        issue_gather(1)
        compute_scatter(0)
        issue_loads(i + 2, 0)
        issue_gather(0)
        compute_scatter(1)

        @pl.when(i + 3 < NWIN)
        def _():
            issue_loads(i + 3, 1)

        return carry

    lax.fori_loop(0, NWIN // 2, pair, 0)
    compute_scatter(0)

    plsc.subcore_barrier()
    for k in range(RPT // ZR):
        rbase = sid * RPT + k * ZR
        pltpu.sync_copy(accum.at[pl.ds(rbase, ZR)],
                        out_hbm.at[cid, pl.ds(rbase, ZR)])


@functools.partial(
    pl.kernel,
    out_type=jax.ShapeDtypeStruct((NC, NPAD, H), FP32),
    mesh=_SC_MESH,
    scratch_types=[
        pltpu.VMEM((W,), jnp.int32),
        pltpu.VMEM((W,), jnp.int32),
        pltpu.VMEM((W, H), FP32),
        pltpu.VMEM((W // 2, H), jnp.int32),
        pltpu.VMEM((W,), jnp.int32),
        pltpu.VMEM((W,), jnp.int32),
        pltpu.VMEM((W, H), FP32),
        pltpu.VMEM((W // 2, H), jnp.int32),
        pltpu.VMEM((W, H), FP32),
        pltpu.VMEM_SHARED((NPAD, H), FP32),
        pltpu.SemaphoreType.DMA,
        pltpu.SemaphoreType.DMA,
        pltpu.SemaphoreType.DMA,
        pltpu.SemaphoreType.DMA,
        pltpu.SemaphoreType.DMA,
        pltpu.SemaphoreType.DMA,
    ],
    compiler_params=pltpu.CompilerParams(needs_layout_passes=False),
)
def _sc_layer(h_hbm, eh_hbm, src_hbm, dst_hbm, out_hbm,
              srcb0, dstb0, hrows0, erows0,
              srcb1, dstb1, hrows1, erows1,
              msg, accum,
              sem_i0, sem_i1, sem_g0, sem_g1, sem_e0, sem_e1):
    _sc_body(h_hbm, eh_hbm, src_hbm, dst_hbm, out_hbm,
             srcb0, dstb0, hrows0, erows0,
             srcb1, dstb1, hrows1, erows1,
             msg, accum,
             sem_i0, sem_i1, sem_g0, sem_g1, sem_e0, sem_e1)


def kernel(x, edge_index, edge_attr, batch, atom_emb, W_edge, b_edge, W1, b1,
           W2, b2, W_ih_gru, W_hh_gru, b_ih_gru, b_hh_gru,
           W_ih_lstm, W_hh_lstm, b_ih_lstm, b_hh_lstm, W_fc, b_fc):
    src = edge_index[0]
    dst = edge_index[1]
    batch2d = batch.reshape(N, 1)
    WihT_g = W_ih_gru.T
    WhhT_g = W_hh_gru.T
    WihT_l = W_ih_lstm.T
    WhhT_l = W_hh_lstm.T
    lo_idx = jnp.asarray(
        [32 * (k // 16) + (k % 16) for k in range(H // 2)], jnp.int32)
    hi_idx = lo_idx + 16
    W_lo = W_edge[:, lo_idx]
    W_hi = W_edge[:, hi_idx]
    b_lo = b_edge[lo_idx].reshape(1, H // 2)
    b_hi = b_edge[hi_idx].reshape(1, H // 2)
    b1_2 = b1.reshape(1, H)
    b2_2 = b2.reshape(1, H)
    bih_g = b_ih_gru.reshape(1, 3 * H)
    bhh_g = b_hh_gru.reshape(1, 3 * H)
    bih_l = b_ih_lstm.reshape(1, 4 * H)
    bhh_l = b_hh_lstm.reshape(1, 4 * H)
    bfc2 = b_fc.reshape(1, 1)

    h = _encode(x, atom_emb)
    edge_h = _edge_mlp(edge_attr.reshape(E // 2, 32), W_lo, b_lo, W_hi, b_hi)
    for _ in range(3):
        partials = _sc_layer(h, edge_h, src, dst)
        h = _dense_layer(h, partials[0], partials[1], W1, b1_2, W2, b2_2,
                         WihT_g, bih_g, WhhT_g, bhh_g)
    return _set2set(h, batch2d, WihT_l, bih_l, WhhT_l, bhh_l, W_fc, bfc2)


# speedup vs baseline: 6.5816x; 1.7163x over previous
import functools

import jax
import jax.numpy as jnp
from jax import lax
from jax.experimental import pallas as pl
from jax.experimental.pallas import tpu as pltpu
from jax.experimental.pallas import tpu_sc as plsc

N = 10000
E = 320000
H = 128
B = 64
NUM_FEAT = 9
VOCAB = 64
EPS = 1e-10

NC = 2
NS = 16
NW = NC * NS
EPW = E // NW
W = 40
NWIN = EPW // W
NPAD = 10240
RPT = NPAD // NS
ZR = 128

FP32 = jnp.float32


def _encode_body(x_ref, emb_ref, o_ref):
    x = x_ref[...]
    iota = lax.broadcasted_iota(jnp.int32, (1, VOCAB), 1)
    acc = jnp.zeros((x.shape[0], H), FP32)
    for f in range(NUM_FEAT):
        oh = (x[:, f:f + 1] == iota).astype(FP32)
        acc = acc + jnp.dot(oh, emb_ref[f], preferred_element_type=FP32)
    o_ref[...] = acc


def _encode(x, atom_emb):
    return pl.pallas_call(
        _encode_body,
        grid=(N // RB,),
        in_specs=[
            pl.BlockSpec((RB, NUM_FEAT), lambda i: (i, 0)),
            pl.BlockSpec((NUM_FEAT, VOCAB, H), lambda i: (0, 0, 0)),
        ],
        out_specs=pl.BlockSpec((RB, H), lambda i: (i, 0)),
        out_shape=jax.ShapeDtypeStruct((N, H), FP32),
    )(x, atom_emb)


EB = 2000


def _edge_body(ea_ref, w_ref, b_ref, o_ref):
    o_ref[...] = (
        jnp.dot(ea_ref[...], w_ref[...], preferred_element_type=FP32)
        + b_ref[...]
    )


def _edge_mlp(edge_attr, W_edge, b_edge2):
    return pl.pallas_call(
        _edge_body,
        grid=(E // EB,),
        in_specs=[
            pl.BlockSpec((EB, 16), lambda i: (i, 0)),
            pl.BlockSpec((16, H), lambda i: (0, 0)),
            pl.BlockSpec((1, H), lambda i: (0, 0)),
        ],
        out_specs=pl.BlockSpec((EB, H), lambda i: (i, 0)),
        out_shape=jax.ShapeDtypeStruct((E, H), FP32),
    )(edge_attr, W_edge, b_edge2)


RB = 1000


def _dense_body(h_ref, p0_ref, p1_ref, w1_ref, b1_ref, w2_ref, b2_ref,
                wih_ref, bih_ref, whh_ref, bhh_ref, o_ref):
    h = h_ref[...]
    z = h + p0_ref[...] + p1_ref[...]
    t = jnp.maximum(
        jnp.dot(z, w1_ref[...], preferred_element_type=FP32) + b1_ref[...], 0.0)
    z2 = jnp.dot(t, w2_ref[...], preferred_element_type=FP32) + b2_ref[...]
    gx = jnp.dot(z2, wih_ref[...], preferred_element_type=FP32) + bih_ref[...]
    gh = jnp.dot(h, whh_ref[...], preferred_element_type=FP32) + bhh_ref[...]
    r = jax.nn.sigmoid(gx[:, :H] + gh[:, :H])
    zg = jax.nn.sigmoid(gx[:, H:2 * H] + gh[:, H:2 * H])
    n = jnp.tanh(gx[:, 2 * H:] + r * gh[:, 2 * H:])
    o_ref[...] = (1.0 - zg) * n + zg * h


def _dense_layer(h, p0, p1, W1, b1, W2, b2, WihT, b_ih, WhhT, b_hh):
    wspec = lambda shape: pl.BlockSpec(shape, lambda i: (0, 0))
    return pl.pallas_call(
        _dense_body,
        grid=(N // RB,),
        in_specs=[
            pl.BlockSpec((RB, H), lambda i: (i, 0)),
            pl.BlockSpec((RB, H), lambda i: (i, 0)),
            pl.BlockSpec((RB, H), lambda i: (i, 0)),
            wspec((H, H)), wspec((1, H)),
            wspec((H, H)), wspec((1, H)),
            wspec((H, 3 * H)), wspec((1, 3 * H)),
            wspec((H, 3 * H)), wspec((1, 3 * H)),
        ],
        out_specs=pl.BlockSpec((RB, H), lambda i: (i, 0)),
        out_shape=jax.ShapeDtypeStruct((N, H), FP32),
    )(h, p0, p1, W1, b1, W2, b2, WihT, b_ih, WhhT, b_hh)


def _s2s_body(h_ref, bt_ref, wih_ref, bih_ref, whh_ref, bhh_ref,
              wfc_ref, bfc_ref, o_ref):
    h = h_ref[...]
    bt = bt_ref[...]
    mask = (bt == lax.broadcasted_iota(jnp.int32, (1, B), 1)).astype(FP32)
    hl = jnp.zeros((B, H), FP32)
    cl = jnp.zeros((B, H), FP32)
    q_star = jnp.zeros((B, 2 * H), FP32)
    for _ in range(3):
        gates = (
            jnp.dot(q_star, wih_ref[...], preferred_element_type=FP32)
            + bih_ref[...]
            + jnp.dot(hl, whh_ref[...], preferred_element_type=FP32)
            + bhh_ref[...]
        )
        ig = jax.nn.sigmoid(gates[:, :H])
        fg = jax.nn.sigmoid(gates[:, H:2 * H])
        gg = jnp.tanh(gates[:, 2 * H:3 * H])
        og = jax.nn.sigmoid(gates[:, 3 * H:])
        cl = fg * cl + ig * gg
        hl = og * jnp.tanh(cl)
        p_all = lax.dot_general(h, hl, (((1,), (1,)), ((), ())),
                                preferred_element_type=FP32)
        prod = jnp.sum(mask * p_all, axis=1, keepdims=True)
        m = jnp.max(jnp.where(mask > 0, p_all, -1e30), axis=0, keepdims=True)
        e = jnp.exp(prod - jnp.sum(mask * m, axis=1, keepdims=True))
        norm = jnp.sum(mask * e, axis=0, keepdims=True)
        att = e / (jnp.sum(mask * norm, axis=1, keepdims=True) + EPS)
        att_out = lax.dot_general(mask * att, h, (((0,), (0,)), ((), ())),
                                  preferred_element_type=FP32)
        q_star = jnp.concatenate([hl, att_out], axis=1)
    o_ref[...] = (
        jnp.dot(q_star, wfc_ref[...], preferred_element_type=FP32)
        + bfc_ref[...]
    )


def _set2set(h, batch2d, WihT, b_ih, WhhT, b_hh, W_fc, b_fc):
    return pl.pallas_call(
        _s2s_body,
        out_shape=jax.ShapeDtypeStruct((B, 1), FP32),
    )(h, batch2d, WihT, b_ih, WhhT, b_hh, W_fc, b_fc)


_SC_MESH = plsc.VectorSubcoreMesh(core_axis_name="c", subcore_axis_name="s")


def _sc_body(h_hbm, eh_hbm, src_hbm, dst_hbm, out_hbm, refs):
    cid = lax.axis_index("c")
    sid = lax.axis_index("s")
    wid = cid * NS + sid

    srcb = refs[0:8]
    dstb = refs[8:16]
    hrows = refs[16:20]
    erows = refs[20:24]
    accum = refs[24]
    sem_i = refs[25:33]
    sem_g = refs[33:37]
    sem_e = refs[37:41]
    sem_s = refs[41:45]

    for hs in range(4):
        def zbody(r, carry, hs=hs):
            for c in range(8):
                hrows[hs][r, pl.ds(c * 16, 16)] = jnp.zeros((16,), FP32)
            return carry
        lax.fori_loop(0, W, zbody, 0)
    for isl in range(4):
        dstb[isl][pl.ds(0, 16)] = jnp.zeros((16,), jnp.int32)
        dstb[isl][pl.ds(16, 16)] = jnp.zeros((16,), jnp.int32)
        dstb[isl][pl.ds(W - 16, 16)] = jnp.zeros((16,), jnp.int32)
    for k in range(RPT // W):
        pltpu.sync_copy(hrows[0], accum.at[pl.ds(sid * RPT + k * W, W)])
    plsc.subcore_barrier()

    for hs in range(4):
        pltpu.async_copy(hrows[hs], accum.at[dstb[hs]], sem_s[hs], add=True)

    ebase = wid * EPW

    def c_loads(i, es, isl):
        base = ebase + i * W
        pltpu.async_copy(src_hbm.at[pl.ds(base, W)], srcb[isl], sem_i[isl])
        pltpu.async_copy(dst_hbm.at[pl.ds(base, W)], dstb[isl], sem_i[isl])
        pltpu.async_copy(eh_hbm.at[pl.ds(base, W)], erows[es], sem_e[es])

    def a_gather(hs, isl):
        pltpu.make_async_copy(src_hbm.at[pl.ds(0, W)], srcb[isl],
                              sem_i[isl]).wait()
        pltpu.make_async_copy(dst_hbm.at[pl.ds(0, W)], dstb[isl],
                              sem_i[isl]).wait()
        pltpu.make_async_copy(hrows[hs], accum.at[dstb[isl]],
                              sem_s[hs]).wait()
        pltpu.async_copy(h_hbm.at[srcb[isl]], hrows[hs], sem_g[hs])

    def b_comp(hs, isl):
        pltpu.make_async_copy(h_hbm.at[srcb[isl]], hrows[hs],
                              sem_g[hs]).wait()
        pltpu.make_async_copy(eh_hbm.at[pl.ds(0, W)], erows[hs],
                              sem_e[hs]).wait()
        hr = hrows[hs]
        er = erows[hs]

        def erow(e2, carry2):
            for c in range(8):
                hv = hr[e2, pl.ds(c * 16, 16)]
                ev = er[e2, pl.ds(c * 16, 16)]
                hr[e2, pl.ds(c * 16, 16)] = jnp.maximum(hv + ev, 0.0)
            return carry2

        lax.fori_loop(0, W, erow, 0)
        pltpu.async_copy(hr, accum.at[dstb[isl]], sem_s[hs], add=True)

    c_loads(0, 0, 0)
    c_loads(1, 1, 1)
    c_loads(2, 2, 2)
    a_gather(0, 0)
    a_gather(1, 1)

    TAIL = 10
    BODY = NWIN - TAIL

    def oct_body(q, carry):
        i = 8 * q
        for k in range(8):
            c_loads(i + k + 3, (k + 3) % 4, (k + 3) % 8)
            a_gather((k + 2) % 4, (k + 2) % 8)
            b_comp(k % 4, k % 8)
        return carry

    lax.fori_loop(0, BODY // 8, oct_body, 0)
    for i in range(BODY, NWIN):
        if i + 3 < NWIN:
            c_loads(i + 3, (i + 3) % 4, (i + 3) % 8)
        if i + 2 < NWIN:
            a_gather((i + 2) % 4, (i + 2) % 8)
        b_comp(i % 4, i % 8)
    for hs in range(4):
        pltpu.make_async_copy(hrows[hs], accum.at[dstb[hs]],
                              sem_s[hs]).wait()

    plsc.subcore_barrier()
    for k in range(RPT // ZR):
        rbase = sid * RPT + k * ZR
        pltpu.sync_copy(accum.at[pl.ds(rbase, ZR)],
                        out_hbm.at[cid, pl.ds(rbase, ZR)])


@functools.partial(
    pl.kernel,
    out_type=jax.ShapeDtypeStruct((NC, NPAD, H), FP32),
    mesh=_SC_MESH,
    scratch_types=(
        [pltpu.VMEM((W,), jnp.int32)] * 16
        + [pltpu.VMEM((W, H), FP32)] * 4
        + [pltpu.VMEM((W, H), FP32)] * 4
        + [pltpu.VMEM_SHARED((NPAD, H), FP32)]
        + [pltpu.SemaphoreType.DMA] * 20
    ),
    compiler_params=pltpu.CompilerParams(needs_layout_passes=False),
)
def _sc_layer(h_hbm, eh_hbm, src_hbm, dst_hbm, out_hbm, *refs):
    _sc_body(h_hbm, eh_hbm, src_hbm, dst_hbm, out_hbm, refs)


def kernel(x, edge_index, edge_attr, batch, atom_emb, W_edge, b_edge, W1, b1,
           W2, b2, W_ih_gru, W_hh_gru, b_ih_gru, b_hh_gru,
           W_ih_lstm, W_hh_lstm, b_ih_lstm, b_hh_lstm, W_fc, b_fc):
    src = edge_index[0]
    dst = edge_index[1]
    batch2d = batch.reshape(N, 1)
    WihT_g = W_ih_gru.T
    WhhT_g = W_hh_gru.T
    WihT_l = W_ih_lstm.T
    WhhT_l = W_hh_lstm.T
    b_edge2 = b_edge.reshape(1, H)
    b1_2 = b1.reshape(1, H)
    b2_2 = b2.reshape(1, H)
    bih_g = b_ih_gru.reshape(1, 3 * H)
    bhh_g = b_hh_gru.reshape(1, 3 * H)
    bih_l = b_ih_lstm.reshape(1, 4 * H)
    bhh_l = b_hh_lstm.reshape(1, 4 * H)
    bfc2 = b_fc.reshape(1, 1)

    h = _encode(x, atom_emb)
    edge_h = _edge_mlp(edge_attr, W_edge, b_edge2)
    for _ in range(3):
        partials = _sc_layer(h, edge_h, src, dst)
        h = _dense_layer(h, partials[0], partials[1], W1, b1_2, W2, b2_2,
                         WihT_g, bih_g, WhhT_g, bhh_g)
    return _set2set(h, batch2d, WihT_l, bih_l, WhhT_l, bhh_l, W_fc, bfc2)


# speedup vs baseline: 7.9614x; 1.2096x over previous
import functools

import jax
import jax.numpy as jnp
from jax import lax
from jax.experimental import pallas as pl
from jax.experimental.pallas import tpu as pltpu
from jax.experimental.pallas import tpu_sc as plsc

N = 10000
E = 320000
H = 128
B = 64
NUM_FEAT = 9
VOCAB = 64
EPS = 1e-10

NC = 2
NS = 16
NW = NC * NS
EPW = E // NW
W = 40
NWIN = EPW // W
NPAD = 10240
RPT = NPAD // NS
ZR = 128

FP32 = jnp.float32


def _encode_body(x_ref, emb_ref, o_ref):
    x = x_ref[...]
    iota = lax.broadcasted_iota(jnp.int32, (1, VOCAB), 1)
    acc = jnp.zeros((x.shape[0], H), FP32)
    for f in range(NUM_FEAT):
        oh = (x[:, f:f + 1] == iota).astype(FP32)
        acc = acc + jnp.dot(oh, emb_ref[f], preferred_element_type=FP32)
    o_ref[...] = acc


def _encode(x, atom_emb):
    return pl.pallas_call(
        _encode_body,
        grid=(N // RB,),
        in_specs=[
            pl.BlockSpec((RB, NUM_FEAT), lambda i: (i, 0)),
            pl.BlockSpec((NUM_FEAT, VOCAB, H), lambda i: (0, 0, 0)),
        ],
        out_specs=pl.BlockSpec((RB, H), lambda i: (i, 0)),
        out_shape=jax.ShapeDtypeStruct((N, H), FP32),
    )(x, atom_emb)


EB = 2560


def _edge_body(eat_ref, w_ref, b_ref, o_ref):
    o_ref[...] = (
        lax.dot_general(eat_ref[...], w_ref[...], (((0,), (0,)), ((), ())),
                        preferred_element_type=FP32)
        + b_ref[...]
    )


def _edge_mlp(edge_attr_t, W_edge, b_edge2):
    return pl.pallas_call(
        _edge_body,
        grid=(E // EB,),
        in_specs=[
            pl.BlockSpec((16, EB), lambda i: (0, i)),
            pl.BlockSpec((16, H), lambda i: (0, 0)),
            pl.BlockSpec((1, H), lambda i: (0, 0)),
        ],
        out_specs=pl.BlockSpec((EB, H), lambda i: (i, 0)),
        out_shape=jax.ShapeDtypeStruct((E, H), FP32),
    )(edge_attr_t, W_edge, b_edge2)


RB = 1000


def _dense_body(h_ref, p0_ref, p1_ref, w1_ref, b1_ref, w2_ref, b2_ref,
                wih_ref, bih_ref, whh_ref, bhh_ref, o_ref):
    h = h_ref[...]
    z = h + p0_ref[0] + p1_ref[0]
    t = jnp.maximum(
        jnp.dot(z, w1_ref[...], preferred_element_type=FP32) + b1_ref[...], 0.0)
    z2 = jnp.dot(t, w2_ref[...], preferred_element_type=FP32) + b2_ref[...]
    gx = jnp.dot(z2, wih_ref[...], preferred_element_type=FP32) + bih_ref[...]
    gh = jnp.dot(h, whh_ref[...], preferred_element_type=FP32) + bhh_ref[...]
    r = jax.nn.sigmoid(gx[:, :H] + gh[:, :H])
    zg = jax.nn.sigmoid(gx[:, H:2 * H] + gh[:, H:2 * H])
    n = jnp.tanh(gx[:, 2 * H:] + r * gh[:, 2 * H:])
    o_ref[...] = (1.0 - zg) * n + zg * h


def _dense_layer(h, partials, W1, b1, W2, b2, WihT, b_ih, WhhT, b_hh):
    wspec = lambda shape: pl.BlockSpec(shape, lambda i: (0, 0))
    return pl.pallas_call(
        _dense_body,
        grid=(N // RB,),
        in_specs=[
            pl.BlockSpec((RB, H), lambda i: (i, 0)),
            pl.BlockSpec((1, RB, H), lambda i: (0, i, 0)),
            pl.BlockSpec((1, RB, H), lambda i: (1, i, 0)),
            wspec((H, H)), wspec((1, H)),
            wspec((H, H)), wspec((1, H)),
            wspec((H, 3 * H)), wspec((1, 3 * H)),
            wspec((H, 3 * H)), wspec((1, 3 * H)),
        ],
        out_specs=pl.BlockSpec((RB, H), lambda i: (i, 0)),
        out_shape=jax.ShapeDtypeStruct((N, H), FP32),
    )(h, partials, partials, W1, b1, W2, b2, WihT, b_ih, WhhT, b_hh)


def _s2s_body(h_ref, bt_ref, wih_ref, bih_ref, whh_ref, bhh_ref,
              wfc_ref, bfc_ref, o_ref):
    h = h_ref[...]
    bt = bt_ref[...]
    mask = (bt == lax.broadcasted_iota(jnp.int32, (1, B), 1)).astype(FP32)
    hl = jnp.zeros((B, H), FP32)
    cl = jnp.zeros((B, H), FP32)
    q_star = jnp.zeros((B, 2 * H), FP32)
    for _ in range(3):
        gates = (
            jnp.dot(q_star, wih_ref[...], preferred_element_type=FP32)
            + bih_ref[...]
            + jnp.dot(hl, whh_ref[...], preferred_element_type=FP32)
            + bhh_ref[...]
        )
        ig = jax.nn.sigmoid(gates[:, :H])
        fg = jax.nn.sigmoid(gates[:, H:2 * H])
        gg = jnp.tanh(gates[:, 2 * H:3 * H])
        og = jax.nn.sigmoid(gates[:, 3 * H:])
        cl = fg * cl + ig * gg
        hl = og * jnp.tanh(cl)
        p_all = lax.dot_general(h, hl, (((1,), (1,)), ((), ())),
                                preferred_element_type=FP32)
        prod = jnp.sum(mask * p_all, axis=1, keepdims=True)
        m = jnp.max(jnp.where(mask > 0, p_all, -1e30), axis=0, keepdims=True)
        e = jnp.exp(prod - jnp.sum(mask * m, axis=1, keepdims=True))
        norm = jnp.sum(mask * e, axis=0, keepdims=True)
        att = e / (jnp.sum(mask * norm, axis=1, keepdims=True) + EPS)
        att_out = lax.dot_general(mask * att, h, (((0,), (0,)), ((), ())),
                                  preferred_element_type=FP32)
        q_star = jnp.concatenate([hl, att_out], axis=1)
    o_ref[...] = (
        jnp.dot(q_star, wfc_ref[...], preferred_element_type=FP32)
        + bfc_ref[...]
    )


def _set2set(h, batch2d, WihT, b_ih, WhhT, b_hh, W_fc, b_fc):
    return pl.pallas_call(
        _s2s_body,
        out_shape=jax.ShapeDtypeStruct((B, 1), FP32),
    )(h, batch2d, WihT, b_ih, WhhT, b_hh, W_fc, b_fc)


_SC_MESH = plsc.VectorSubcoreMesh(core_axis_name="c", subcore_axis_name="s")


def _sc_body(h_hbm, eh_hbm, src_hbm, dst_hbm, out_hbm, refs):
    cid = lax.axis_index("c")
    sid = lax.axis_index("s")
    wid = cid * NS + sid

    srcb = refs[0:8]
    dstb = refs[8:16]
    hrows = refs[16:20]
    erows = refs[20:24]
    accum = refs[24]
    sem_i = refs[25:33]
    sem_g = refs[33:37]
    sem_e = refs[37:41]
    sem_s = refs[41:45]

    for hs in range(4):
        def zbody(r, carry, hs=hs):
            for c in range(8):
                hrows[hs][r, pl.ds(c * 16, 16)] = jnp.zeros((16,), FP32)
            return carry
        lax.fori_loop(0, W, zbody, 0)
    for isl in range(4):
        dstb[isl][pl.ds(0, 16)] = jnp.zeros((16,), jnp.int32)
        dstb[isl][pl.ds(16, 16)] = jnp.zeros((16,), jnp.int32)
        dstb[isl][pl.ds(W - 16, 16)] = jnp.zeros((16,), jnp.int32)
    for k in range(RPT // W):
        pltpu.sync_copy(hrows[0], accum.at[pl.ds(sid * RPT + k * W, W)])
    plsc.subcore_barrier()

    for hs in range(4):
        pltpu.async_copy(hrows[hs], accum.at[dstb[hs]], sem_s[hs], add=True)

    ebase = wid * EPW

    def c_loads(i, es, isl):
        base = ebase + i * W
        pltpu.async_copy(src_hbm.at[pl.ds(base, W)], srcb[isl], sem_i[isl])
        pltpu.async_copy(dst_hbm.at[pl.ds(base, W)], dstb[isl], sem_i[isl])
        pltpu.async_copy(eh_hbm.at[pl.ds(base, W)], erows[es], sem_e[es])

    def a_gather(hs, isl):
        pltpu.make_async_copy(src_hbm.at[pl.ds(0, W)], srcb[isl],
                              sem_i[isl]).wait()
        pltpu.make_async_copy(dst_hbm.at[pl.ds(0, W)], dstb[isl],
                              sem_i[isl]).wait()
        pltpu.make_async_copy(hrows[hs], accum.at[dstb[isl]],
                              sem_s[hs]).wait()
        pltpu.async_copy(h_hbm.at[srcb[isl]], hrows[hs], sem_g[hs])

    def b_comp(hs, isl):
        pltpu.make_async_copy(h_hbm.at[srcb[isl]], hrows[hs],
                              sem_g[hs]).wait()
        pltpu.make_async_copy(eh_hbm.at[pl.ds(0, W)], erows[hs],
                              sem_e[hs]).wait()
        hr = hrows[hs]
        er = erows[hs]

        def erow(e2, carry2):
            for c in range(8):
                hv = hr[e2, pl.ds(c * 16, 16)]
                ev = er[e2, pl.ds(c * 16, 16)]
                hr[e2, pl.ds(c * 16, 16)] = jnp.maximum(hv + ev, 0.0)
            return carry2

        lax.fori_loop(0, W, erow, 0)
        pltpu.async_copy(hr, accum.at[dstb[isl]], sem_s[hs], add=True)

    c_loads(0, 0, 0)
    c_loads(1, 1, 1)
    c_loads(2, 2, 2)
    a_gather(0, 0)
    a_gather(1, 1)

    TAIL = 10
    BODY = NWIN - TAIL

    def oct_body(q, carry):
        i = 8 * q
        for k in range(8):
            c_loads(i + k + 3, (k + 3) % 4, (k + 3) % 8)
            a_gather((k + 2) % 4, (k + 2) % 8)
            b_comp(k % 4, k % 8)
        return carry

    lax.fori_loop(0, BODY // 8, oct_body, 0)
    for i in range(BODY, NWIN):
        if i + 3 < NWIN:
            c_loads(i + 3, (i + 3) % 4, (i + 3) % 8)
        if i + 2 < NWIN:
            a_gather((i + 2) % 4, (i + 2) % 8)
        b_comp(i % 4, i % 8)
    for hs in range(4):
        pltpu.make_async_copy(hrows[hs], accum.at[dstb[hs]],
                              sem_s[hs]).wait()

    plsc.subcore_barrier()
    for k in range(RPT // ZR):
        rbase = sid * RPT + k * ZR
        pltpu.sync_copy(accum.at[pl.ds(rbase, ZR)],
                        out_hbm.at[cid, pl.ds(rbase, ZR)])


@functools.partial(
    pl.kernel,
    out_type=jax.ShapeDtypeStruct((NC, NPAD, H), FP32),
    mesh=_SC_MESH,
    scratch_types=(
        [pltpu.VMEM((W,), jnp.int32)] * 16
        + [pltpu.VMEM((W, H), FP32)] * 4
        + [pltpu.VMEM((W, H), FP32)] * 4
        + [pltpu.VMEM_SHARED((NPAD, H), FP32)]
        + [pltpu.SemaphoreType.DMA] * 20
    ),
    compiler_params=pltpu.CompilerParams(needs_layout_passes=False),
)
def _sc_layer(h_hbm, eh_hbm, src_hbm, dst_hbm, out_hbm, *refs):
    _sc_body(h_hbm, eh_hbm, src_hbm, dst_hbm, out_hbm, refs)


def kernel(x, edge_index, edge_attr, batch, atom_emb, W_edge, b_edge, W1, b1,
           W2, b2, W_ih_gru, W_hh_gru, b_ih_gru, b_hh_gru,
           W_ih_lstm, W_hh_lstm, b_ih_lstm, b_hh_lstm, W_fc, b_fc):
    src = edge_index[0]
    dst = edge_index[1]
    batch2d = batch.reshape(N, 1)
    WihT_g = W_ih_gru.T
    WhhT_g = W_hh_gru.T
    WihT_l = W_ih_lstm.T
    WhhT_l = W_hh_lstm.T
    b_edge2 = b_edge.reshape(1, H)
    b1_2 = b1.reshape(1, H)
    b2_2 = b2.reshape(1, H)
    bih_g = b_ih_gru.reshape(1, 3 * H)
    bhh_g = b_hh_gru.reshape(1, 3 * H)
    bih_l = b_ih_lstm.reshape(1, 4 * H)
    bhh_l = b_hh_lstm.reshape(1, 4 * H)
    bfc2 = b_fc.reshape(1, 1)

    h = _encode(x, atom_emb)
    edge_h = _edge_mlp(edge_attr.T, W_edge, b_edge2)
    for _ in range(3):
        partials = _sc_layer(h, edge_h, src, dst)
        h = _dense_layer(h, partials, W1, b1_2, W2, b2_2,
                         WihT_g, bih_g, WhhT_g, bhh_g)
    return _set2set(h, batch2d, WihT_l, bih_l, WhhT_l, bhh_l, W_fc, bfc2)
